# Initial kernel scaffold; baseline (speedup 1.0000x reference)
import functools

import jax
import jax.numpy as jnp
from jax import lax
from jax.experimental import pallas as pl
from jax.experimental.pallas import tpu as pltpu
from jax.experimental.pallas import tpu_sc as plsc

_LANES = 16


def _encode_body(x_ref, w_ref, b_ref, y_ref):
    acc = jnp.dot(x_ref[...], w_ref[...], preferred_element_type=jnp.float32)
    y_ref[...] = jnp.tanh(acc + b_ref[...])


def _decode_loss_body(x_ref, y_ref, w_ref, b_ref, out_ref):
    xh = jnp.dot(y_ref[...], w_ref[...], preferred_element_type=jnp.float32)
    d = x_ref[...] - (xh + b_ref[...])
    out_ref[0, 0] = jnp.sum(d * d)


def _make_edge_kernel(n_workers, n_chunks, chunk, h):
    mesh = plsc.VectorSubcoreMesh(core_axis_name="core", subcore_axis_name="sub")
    info = plsc.get_sparse_core_info()
    nc = info.num_cores

    @functools.partial(
        pl.kernel,
        mesh=mesh,
        out_type=jax.ShapeDtypeStruct((n_workers, _LANES), jnp.float32),
        scratch_types=[
            pltpu.VMEM((n_chunks, chunk), jnp.int32),
            pltpu.VMEM((n_chunks, chunk), jnp.int32),
            pltpu.VMEM((n_chunks, chunk), jnp.float32),
            pltpu.VMEM((chunk, h), jnp.float32),
            pltpu.VMEM((chunk, h), jnp.float32),
            pltpu.VMEM((_LANES,), jnp.float32),
            pltpu.SemaphoreType.DMA,
            pltpu.SemaphoreType.DMA,
        ],
    )
    def edge_kernel(row_hbm, col_hbm, wgt_hbm, y_hbm, out_hbm,
                    row_v, col_v, w_v, r1_v, r2_v, out_v, sem1, sem2):
        wid = lax.axis_index("sub") * nc + lax.axis_index("core")
        pltpu.sync_copy(row_hbm.at[wid], row_v)
        pltpu.sync_copy(col_hbm.at[wid], col_v)
        pltpu.sync_copy(wgt_hbm.at[wid], w_v)

        def chunk_body(c, total):
            cp1 = pltpu.async_copy(y_hbm.at[row_v.at[c]], r1_v, sem1)
            cp2 = pltpu.async_copy(y_hbm.at[col_v.at[c]], r2_v, sem2)
            cp1.wait()
            cp2.wait()
            for g in range(chunk // _LANES):
                e_idx = jnp.arange(_LANES, dtype=jnp.int32) + (g * _LANES)
                acc = jnp.zeros((_LANES,), jnp.float32)
                for dd in range(h):
                    d_idx = jnp.full((_LANES,), dd, jnp.int32)
                    v1 = plsc.load_gather(r1_v, [e_idx, d_idx])
                    v2 = plsc.load_gather(r2_v, [e_idx, d_idx])
                    df = v1 - v2
                    acc = acc + df * df
                xx = acc + jnp.float32(1e-12)
                bits = plsc.bitcast(xx, jnp.int32)
                bits = jnp.int32(0x5F3759DF) - lax.shift_right_arithmetic(
                    bits, jnp.int32(1))
                y = plsc.bitcast(bits, jnp.float32)
                for _ in range(3):
                    y = y * (jnp.float32(1.5)
                             - jnp.float32(0.5) * xx * y * y)
                dist = xx * y
                w = w_v[c, pl.ds(g * _LANES, _LANES)]
                total = total + dist * w
            return total

        total = lax.fori_loop(0, n_chunks, chunk_body,
                              jnp.zeros((_LANES,), jnp.float32))
        out_v[...] = total
        pltpu.sync_copy(out_v, out_hbm.at[wid])

    return edge_kernel


def kernel(x, edge_index, edge_weight, W_enc, b_enc, W_dec, b_dec):
    n, d = x.shape
    h = W_enc.shape[1]
    e = edge_weight.shape[0]

    info = plsc.get_sparse_core_info()
    n_workers = info.num_cores * info.num_subcores
    per_worker = e // n_workers
    chunk = 80
    n_chunks = per_worker // chunk
    assert per_worker * n_workers == e and n_chunks * chunk == per_worker

    Y = pl.pallas_call(
        _encode_body,
        out_shape=jax.ShapeDtypeStruct((n, h), jnp.float32),
    )(x, W_enc, b_enc.reshape(1, h))

    row3 = edge_index[0].reshape(n_workers, n_chunks, chunk)
    col3 = edge_index[1].reshape(n_workers, n_chunks, chunk)
    wgt3 = edge_weight.reshape(n_workers, n_chunks, chunk)

    edge_kernel = _make_edge_kernel(n_workers, n_chunks, chunk, h)
    partials = edge_kernel(row3, col3, wgt3, Y)

    sq_sum = pl.pallas_call(
        _decode_loss_body,
        out_shape=jax.ShapeDtypeStruct((1, 1), jnp.float32),
    )(x, Y, W_dec, b_dec.reshape(1, d))

    loss1 = sq_sum[0, 0] / jnp.float32(n * d)
    loss2 = jnp.sum(partials) / jnp.float32(e)
    return loss1 + loss2


# speedup vs baseline: 3.5622x; 3.5622x over previous
import functools

import jax
import jax.numpy as jnp
from jax import lax
from jax.experimental import pallas as pl
from jax.experimental.pallas import tpu as pltpu
from jax.experimental.pallas import tpu_sc as plsc

_LANES = 16


def _encode_body(x_ref, w_ref, b_ref, y_ref):
    acc = jnp.dot(x_ref[...], w_ref[...], preferred_element_type=jnp.float32)
    y_ref[...] = jnp.tanh(acc + b_ref[...])


def _decode_loss_body(x_ref, y_ref, w_ref, b_ref, out_ref):
    xh = jnp.dot(y_ref[...], w_ref[...], preferred_element_type=jnp.float32)
    d = x_ref[...] - (xh + b_ref[...])
    out_ref[...] = jnp.sum(d * d).reshape(1, 1)


def _make_edge_kernel(n_workers, n_chunks, chunk, h):
    mesh = plsc.VectorSubcoreMesh(core_axis_name="core", subcore_axis_name="sub")
    info = plsc.get_sparse_core_info()
    nc = info.num_cores

    @functools.partial(
        pl.kernel,
        mesh=mesh,
        out_type=jax.ShapeDtypeStruct((n_workers, _LANES), jnp.float32),
        scratch_types=[
            pltpu.VMEM((n_chunks, chunk), jnp.int32),
            pltpu.VMEM((n_chunks, chunk), jnp.int32),
            pltpu.VMEM((n_chunks, chunk), jnp.float32),
            pltpu.VMEM((chunk, h), jnp.float32),
            pltpu.VMEM((chunk, h), jnp.float32),
            pltpu.VMEM((_LANES,), jnp.float32),
            pltpu.SemaphoreType.DMA,
            pltpu.SemaphoreType.DMA,
        ],
        compiler_params=pltpu.CompilerParams(
            needs_layout_passes=False, use_tc_tiling_on_sc=False),
    )
    def edge_kernel(row_hbm, col_hbm, wgt_hbm, y_hbm, out_hbm,
                    row_v, col_v, w_v, r1_v, r2_v, out_v, sem1, sem2):
        wid = lax.axis_index("sub") * nc + lax.axis_index("core")
        pltpu.sync_copy(row_hbm.at[wid], row_v)
        pltpu.sync_copy(col_hbm.at[wid], col_v)
        pltpu.sync_copy(wgt_hbm.at[wid], w_v)

        def chunk_body(c, total):
            cp1 = pltpu.async_copy(y_hbm.at[row_v.at[c]], r1_v, sem1)
            cp2 = pltpu.async_copy(y_hbm.at[col_v.at[c]], r2_v, sem2)
            cp1.wait()
            cp2.wait()
            for g in range(chunk // _LANES):
                e_idx = jnp.arange(_LANES, dtype=jnp.int32) + (g * _LANES)
                acc = jnp.zeros((_LANES,), jnp.float32)
                for dd in range(h):
                    d_idx = jnp.full((_LANES,), dd, jnp.int32)
                    v1 = plsc.load_gather(r1_v, [e_idx, d_idx])
                    v2 = plsc.load_gather(r2_v, [e_idx, d_idx])
                    df = v1 - v2
                    acc = acc + df * df
                xx = acc + jnp.float32(1e-12)
                bits = plsc.bitcast(xx, jnp.int32)
                bits = jnp.int32(0x5F3759DF) - lax.shift_right_arithmetic(
                    bits, jnp.int32(1))
                y = plsc.bitcast(bits, jnp.float32)
                for _ in range(3):
                    y = y * (jnp.float32(1.5)
                             - jnp.float32(0.5) * xx * y * y)
                dist = xx * y
                w = w_v[c, pl.ds(g * _LANES, _LANES)]
                total = total + dist * w
            return total

        total = lax.fori_loop(0, n_chunks, chunk_body,
                              jnp.zeros((_LANES,), jnp.float32))
        out_v[...] = total
        pltpu.sync_copy(out_v, out_hbm.at[wid])

    return edge_kernel


def kernel(x, edge_index, edge_weight, W_enc, b_enc, W_dec, b_dec):
    n, d = x.shape
    h = W_enc.shape[1]
    e = edge_weight.shape[0]

    info = plsc.get_sparse_core_info()
    n_workers = info.num_cores * info.num_subcores
    per_worker = e // n_workers
    chunk = 80
    n_chunks = per_worker // chunk
    assert per_worker * n_workers == e and n_chunks * chunk == per_worker

    Y = pl.pallas_call(
        _encode_body,
        out_shape=jax.ShapeDtypeStruct((n, h), jnp.float32),
    )(x, W_enc, b_enc.reshape(1, h))

    row3 = edge_index[0].reshape(n_workers, n_chunks, chunk)
    col3 = edge_index[1].reshape(n_workers, n_chunks, chunk)
    wgt3 = edge_weight.reshape(n_workers, n_chunks, chunk)

    edge_kernel = _make_edge_kernel(n_workers, n_chunks, chunk, h)
    partials = edge_kernel(row3, col3, wgt3, Y)

    sq_sum = pl.pallas_call(
        _decode_loss_body,
        out_shape=jax.ShapeDtypeStruct((1, 1), jnp.float32),
    )(x, Y, W_dec, b_dec.reshape(1, d))

    loss1 = sq_sum[0, 0] / jnp.float32(n * d)
    loss2 = jnp.sum(partials) / jnp.float32(e)
    return loss1 + loss2


# speedup vs baseline: 8.1668x; 2.2927x over previous
import functools

import jax
import jax.numpy as jnp
from jax import lax
from jax.experimental import pallas as pl
from jax.experimental.pallas import tpu as pltpu
from jax.experimental.pallas import tpu_sc as plsc

_LANES = 16


def _encode_body(x_ref, w_ref, b_ref, y_ref):
    acc = jnp.dot(x_ref[...], w_ref[...], preferred_element_type=jnp.float32)
    y_ref[...] = jnp.tanh(acc + b_ref[...])


def _decode_loss_body(x_ref, y_ref, w_ref, b_ref, out_ref):
    xh = jnp.dot(y_ref[...], w_ref[...], preferred_element_type=jnp.float32)
    d = x_ref[...] - (xh + b_ref[...])
    out_ref[...] = jnp.sum(d * d).reshape(1, 1)


def _make_edge_kernel(n_workers, n_chunks, chunk, h):
    mesh = plsc.VectorSubcoreMesh(core_axis_name="core", subcore_axis_name="sub")
    info = plsc.get_sparse_core_info()
    nc = info.num_cores

    @functools.partial(
        pl.kernel,
        mesh=mesh,
        out_type=jax.ShapeDtypeStruct((n_workers, _LANES), jnp.float32),
        scratch_types=[
            pltpu.VMEM((n_chunks, chunk), jnp.int32),
            pltpu.VMEM((n_chunks, chunk), jnp.int32),
            pltpu.VMEM((n_chunks, chunk), jnp.float32),
            pltpu.VMEM((chunk, h), jnp.float32),
            pltpu.VMEM((chunk, h), jnp.float32),
            pltpu.VMEM((_LANES,), jnp.float32),
            pltpu.SemaphoreType.DMA,
            pltpu.SemaphoreType.DMA,
        ],
        compiler_params=pltpu.CompilerParams(
            needs_layout_passes=False, use_tc_tiling_on_sc=False),
    )
    def edge_kernel(row_hbm, col_hbm, wgt_hbm, y_hbm, out_hbm,
                    row_v, col_v, w_v, r1_v, r2_v, out_v, sem1, sem2):
        wid = lax.axis_index("sub") * nc + lax.axis_index("core")
        pltpu.sync_copy(row_hbm.at[wid], row_v)
        pltpu.sync_copy(col_hbm.at[wid], col_v)
        pltpu.sync_copy(wgt_hbm.at[wid], w_v)

        def chunk_body(c, total):
            cp1 = pltpu.async_copy(y_hbm.at[row_v.at[c]], r1_v, sem1)
            cp2 = pltpu.async_copy(y_hbm.at[col_v.at[c]], r2_v, sem2)
            cp1.wait()
            cp2.wait()
            lanes = jnp.arange(_LANES, dtype=jnp.int32)
            for g in range(chunk // _LANES):
                e_idx = lanes + (g * _LANES)
                acc = jnp.zeros((_LANES,), jnp.float32)
                for d0 in range(h):
                    d_idx = (lanes + d0) & (h - 1)
                    v1 = plsc.load_gather(r1_v, [e_idx, d_idx])
                    v2 = plsc.load_gather(r2_v, [e_idx, d_idx])
                    df = v1 - v2
                    acc = acc + df * df
                xx = acc + jnp.float32(1e-12)
                bits = plsc.bitcast(xx, jnp.int32)
                bits = jnp.int32(0x5F3759DF) - lax.shift_right_arithmetic(
                    bits, jnp.int32(1))
                y = plsc.bitcast(bits, jnp.float32)
                for _ in range(3):
                    y = y * (jnp.float32(1.5)
                             - jnp.float32(0.5) * xx * y * y)
                dist = xx * y
                w = w_v[c, pl.ds(g * _LANES, _LANES)]
                total = total + dist * w
            return total

        total = lax.fori_loop(0, n_chunks, chunk_body,
                              jnp.zeros((_LANES,), jnp.float32))
        out_v[...] = total
        pltpu.sync_copy(out_v, out_hbm.at[wid])

    return edge_kernel


def kernel(x, edge_index, edge_weight, W_enc, b_enc, W_dec, b_dec):
    n, d = x.shape
    h = W_enc.shape[1]
    e = edge_weight.shape[0]

    info = plsc.get_sparse_core_info()
    n_workers = info.num_cores * info.num_subcores
    per_worker = e // n_workers
    chunk = 80
    n_chunks = per_worker // chunk
    assert per_worker * n_workers == e and n_chunks * chunk == per_worker

    Y = pl.pallas_call(
        _encode_body,
        out_shape=jax.ShapeDtypeStruct((n, h), jnp.float32),
    )(x, W_enc, b_enc.reshape(1, h))

    row3 = edge_index[0].reshape(n_workers, n_chunks, chunk)
    col3 = edge_index[1].reshape(n_workers, n_chunks, chunk)
    wgt3 = edge_weight.reshape(n_workers, n_chunks, chunk)

    edge_kernel = _make_edge_kernel(n_workers, n_chunks, chunk, h)
    partials = edge_kernel(row3, col3, wgt3, Y)

    sq_sum = pl.pallas_call(
        _decode_loss_body,
        out_shape=jax.ShapeDtypeStruct((1, 1), jnp.float32),
    )(x, Y, W_dec, b_dec.reshape(1, d))

    loss1 = sq_sum[0, 0] / jnp.float32(n * d)
    loss2 = jnp.sum(partials) / jnp.float32(e)
    return loss1 + loss2
